# baseline (speedup 1.0000x reference)
import functools

import jax
import jax.numpy as jnp
from jax import lax
from jax.experimental import pallas as pl
from jax.experimental.pallas import tpu as pltpu
from jax.experimental.pallas import tpu_sc as plsc

_NW = 32
_CHUNK = 19
_GS = (64, 32, 16)
_C = 85


def _sc_body(p0, p1, p2, tg, anch, out,
             tgt_v, anch_v, idx0_v, idx1_v, idx2_v,
             dst0_v, dst1_v, dst2_v, stage_v, sem0, sem1, sem2):
    preds = (p0, p1, p2)
    idxs = (idx0_v, idx1_v, idx2_v)
    dsts = (dst0_v, dst1_v, dst2_v)
    sems = (sem0, sem1, sem2)

    cid = lax.axis_index("c")
    sid = lax.axis_index("s")
    wid = sid * 2 + cid

    pltpu.sync_copy(tg, tgt_v)
    pltpu.sync_copy(anch, anch_v)

    base = wid * _CHUNK
    lane = lax.iota(jnp.int32, 16)

    av, validv = [], []
    bfv, xv, yv, wv, hv = [], [], [], [], []
    for j in range(2):
        p = base + j * 16 + lane
        valid = jnp.logical_and(j * 16 + lane < _CHUNK, p < 600)
        a = lax.div(p, jnp.int32(200))
        n = lax.rem(p, jnp.int32(200))
        av.append(a); validv.append(valid)
        col = lambda c: plsc.load_gather(tgt_v, [n, jnp.full((16,), c, jnp.int32)])
        bfv.append(col(0))
        xv.append(col(2)); yv.append(col(3))
        wv.append(col(4)); hv.append(col(5))

    for l in range(3):
        G = _GS[l]
        gf = jnp.float32(G)
        for j in range(2):
            gi = jnp.clip((xv[j] * gf).astype(jnp.int32), 0, G - 1)
            gj = jnp.clip((yv[j] * gf).astype(jnp.int32), 0, G - 1)
            b = bfv[j].astype(jnp.int32)
            row = ((b * 3 + av[j]) * G + gj) * G + gi
            widx = jnp.where(validv[j], row * _C, 0)
            for c in range(4):
                idxs[l][pl.ds(c * 32 + j * 16, 16)] = widx + c
    cps = [pltpu.async_copy(preds[l].at[idxs[l]], dsts[l], sems[l])
           for l in range(3)]

    for l in range(3):
        G = _GS[l]
        gf = jnp.float32(G)
        cps[l].wait()
        acc_s = jnp.zeros((16,), jnp.float32)
        acc_c = jnp.zeros((16,), jnp.float32)
        for j in range(2):
            gx = xv[j] * gf
            gy = yv[j] * gf
            gw = wv[j] * gf
            gh = hv[j] * gf
            tbx = gx - gx.astype(jnp.int32).astype(jnp.float32)
            tby = gy - gy.astype(jnp.int32).astype(jnp.float32)
            ai = l * 16 + av[j] * 2
            aw = plsc.load_gather(anch_v, [ai])
            ah = plsc.load_gather(anch_v, [ai + 1])
            rw = gw / aw
            rh = gh / ah
            rmax = jnp.maximum(jnp.maximum(rw, 1.0 / rw),
                               jnp.maximum(rh, 1.0 / rh))
            jf = jnp.where(jnp.logical_and(validv[j], rmax < 4.0), 1.0, 0.0)
            ps0 = dsts[l][pl.ds(0 * 32 + j * 16, 16)]
            ps1 = dsts[l][pl.ds(1 * 32 + j * 16, 16)]
            ps2 = dsts[l][pl.ds(2 * 32 + j * 16, 16)]
            ps3 = dsts[l][pl.ds(3 * 32 + j * 16, 16)]
            px = 1.0 / (1.0 + jnp.exp(-ps0))
            py = 1.0 / (1.0 + jnp.exp(-ps1))
            pw = jnp.exp(ps2) * aw
            ph = jnp.exp(ps3) * ah
            iw = jnp.maximum(
                jnp.minimum(px + pw * 0.5, tbx + gw * 0.5)
                - jnp.maximum(px - pw * 0.5, tbx - gw * 0.5), 0.0)
            ih = jnp.maximum(
                jnp.minimum(py + ph * 0.5, tby + gh * 0.5)
                - jnp.maximum(py - ph * 0.5, tby - gh * 0.5), 0.0)
            inter = iw * ih
            union = pw * ph + gw * gh - inter + 1e-7
            iou = inter / union
            acc_s = acc_s + (1.0 - iou) * jf
            acc_c = acc_c + jf
        stage_v[pl.ds(l * 16, 16)] = acc_s
        stage_v[pl.ds((3 + l) * 16, 16)] = acc_c

    pltpu.sync_copy(stage_v, out.at[wid])


@jax.jit
def _sc_partials(p0, p1, p2, targets, anch):
    mesh = plsc.VectorSubcoreMesh(core_axis_name="c", subcore_axis_name="s")
    fn = pl.kernel(
        _sc_body,
        mesh=mesh,
        compiler_params=pltpu.CompilerParams(needs_layout_passes=False),
        out_type=jax.ShapeDtypeStruct((_NW, 96), jnp.float32),
        scratch_types=[
            pltpu.VMEM((200, 6), jnp.float32),
            pltpu.VMEM((48,), jnp.float32),
            pltpu.VMEM((128,), jnp.int32),
            pltpu.VMEM((128,), jnp.int32),
            pltpu.VMEM((128,), jnp.int32),
            pltpu.VMEM((128,), jnp.float32),
            pltpu.VMEM((128,), jnp.float32),
            pltpu.VMEM((128,), jnp.float32),
            pltpu.VMEM((96,), jnp.float32),
            pltpu.SemaphoreType.DMA,
            pltpu.SemaphoreType.DMA,
            pltpu.SemaphoreType.DMA,
        ],
    )
    return fn(p0, p1, p2, targets, anch)


def kernel(pred0, pred1, pred2, targets, anchors0, anchors1, anchors2):
    p0 = pred0.reshape(-1)
    p1 = pred1.reshape(-1)
    p2 = pred2.reshape(-1)
    pad = jnp.ones((10,), jnp.float32)
    anch = jnp.concatenate([
        anchors0.reshape(-1), pad,
        anchors1.reshape(-1), pad,
        anchors2.reshape(-1), pad,
    ])
    parts = _sc_partials(p0, p1, p2, targets, anch)
    t = parts.reshape(_NW, 6, 16).sum(axis=(0, 2))
    lbox = (t[0] / jnp.maximum(t[3], 1.0)
            + t[1] / jnp.maximum(t[4], 1.0)
            + t[2] / jnp.maximum(t[5], 1.0))
    return lbox.astype(jnp.float32)


# speedup vs baseline: 5.1896x; 5.1896x over previous
import functools

import jax
import jax.numpy as jnp
from jax import lax
from jax.experimental import pallas as pl
from jax.experimental.pallas import tpu as pltpu
from jax.experimental.pallas import tpu_sc as plsc

_NW = 32
_CHUNK = 19
_NPAD = 32
_GS = (64, 32, 16)
_C = 85


def _sc_body(p0, p1, p2, tg, anch, out,
             tgt_v, anch_v, idx0_v, idx1_v, idx2_v,
             dst0_v, dst1_v, dst2_v, stage_v, sem0, sem1, sem2):
    preds = (p0, p1, p2)
    idxs = (idx0_v, idx1_v, idx2_v)
    dsts = (dst0_v, dst1_v, dst2_v)
    sems = (sem0, sem1, sem2)

    cid = lax.axis_index("c")
    sid = lax.axis_index("s")
    wid = sid * 2 + cid

    pltpu.sync_copy(tg, tgt_v)
    pltpu.sync_copy(anch, anch_v)

    base = wid * _CHUNK
    lane = lax.iota(jnp.int32, 16)

    av, validv = [], []
    bfv, xv, yv, wv, hv = [], [], [], [], []
    for j in range(2):
        p = base + j * 16 + lane
        valid = jnp.logical_and(j * 16 + lane < _CHUNK, p < 600)
        a = lax.div(p, jnp.int32(200))
        n = lax.rem(p, jnp.int32(200))
        av.append(a); validv.append(valid)
        col = lambda c: plsc.load_gather(tgt_v, [n, jnp.full((16,), c, jnp.int32)])
        bfv.append(col(0))
        xv.append(col(2)); yv.append(col(3))
        wv.append(col(4)); hv.append(col(5))

    rowvs = []
    for l in range(3):
        G = _GS[l]
        rows_l = []
        for j in range(2):
            gi = jnp.clip((xv[j] * float(G)).astype(jnp.int32), 0, G - 1)
            gj = jnp.clip((yv[j] * float(G)).astype(jnp.int32), 0, G - 1)
            b = bfv[j].astype(jnp.int32)
            row = ((b * 3 + av[j]) * G + gj) * G + gi
            rows_l.append(jnp.where(validv[j], row, 0))
        rowvs.append(rows_l)
    cps = []
    for l in range(3):
        for k in range(_CHUNK):
            r = rowvs[l][k // 16][k % 16]
            cps.append(pltpu.async_copy(
                preds[l].at[pl.ds(r, 1), :],
                dsts[l].at[pl.ds(k, 1), :],
                sems[l]))

    for l in range(3):
        G = _GS[l]
        gf = float(G)
        if l == 0:
            for cp in cps:
                cp.wait()
        acc_s = jnp.zeros((16,), jnp.float32)
        acc_c = jnp.zeros((16,), jnp.float32)
        for j in range(2):
            gx = xv[j] * gf
            gy = yv[j] * gf
            gw = wv[j] * gf
            gh = hv[j] * gf
            tbx = gx - gx.astype(jnp.int32).astype(jnp.float32)
            tby = gy - gy.astype(jnp.int32).astype(jnp.float32)
            ai = l * 16 + av[j] * 2
            aw = plsc.load_gather(anch_v, [ai])
            ah = plsc.load_gather(anch_v, [ai + 1])
            rw = gw / aw
            rh = gh / ah
            rmax = jnp.maximum(jnp.maximum(rw, 1.0 / rw),
                               jnp.maximum(rh, 1.0 / rh))
            jf = jnp.where(jnp.logical_and(validv[j], rmax < 4.0), 1.0, 0.0)
            kidx = jnp.minimum(j * 16 + lane, _CHUNK - 1)
            psc = lambda c: plsc.load_gather(
                dsts[l], [kidx, jnp.full((16,), c, jnp.int32)])
            ps0 = psc(0)
            ps1 = psc(1)
            ps2 = psc(2)
            ps3 = psc(3)
            px = 1.0 / (1.0 + jnp.exp(-ps0))
            py = 1.0 / (1.0 + jnp.exp(-ps1))
            pw = jnp.exp(ps2) * aw
            ph = jnp.exp(ps3) * ah
            iw = jnp.maximum(
                jnp.minimum(px + pw * 0.5, tbx + gw * 0.5)
                - jnp.maximum(px - pw * 0.5, tbx - gw * 0.5), 0.0)
            ih = jnp.maximum(
                jnp.minimum(py + ph * 0.5, tby + gh * 0.5)
                - jnp.maximum(py - ph * 0.5, tby - gh * 0.5), 0.0)
            inter = iw * ih
            union = pw * ph + gw * gh - inter + 1e-7
            iou = inter / union
            acc_s = acc_s + (1.0 - iou) * jf
            acc_c = acc_c + jf
        stage_v[pl.ds(l * 16, 16)] = acc_s
        stage_v[pl.ds((3 + l) * 16, 16)] = acc_c

    pltpu.sync_copy(stage_v, out.at[wid])


@jax.jit
def _sc_partials(p0, p1, p2, targets, anch):
    mesh = plsc.VectorSubcoreMesh(core_axis_name="c", subcore_axis_name="s")
    fn = pl.kernel(
        _sc_body,
        mesh=mesh,
        compiler_params=pltpu.CompilerParams(needs_layout_passes=False),
        out_type=jax.ShapeDtypeStruct((_NW, 96), jnp.float32),
        scratch_types=[
            pltpu.VMEM((200, 6), jnp.float32),
            pltpu.VMEM((48,), jnp.float32),
            pltpu.VMEM((_NPAD,), jnp.int32),
            pltpu.VMEM((_NPAD,), jnp.int32),
            pltpu.VMEM((_NPAD,), jnp.int32),
            pltpu.VMEM((_CHUNK, _C), jnp.float32),
            pltpu.VMEM((_CHUNK, _C), jnp.float32),
            pltpu.VMEM((_CHUNK, _C), jnp.float32),
            pltpu.VMEM((96,), jnp.float32),
            pltpu.SemaphoreType.DMA,
            pltpu.SemaphoreType.DMA,
            pltpu.SemaphoreType.DMA,
        ],
    )
    return fn(p0, p1, p2, targets, anch)


def kernel(pred0, pred1, pred2, targets, anchors0, anchors1, anchors2):
    p0 = pred0.reshape(-1, _C)
    p1 = pred1.reshape(-1, _C)
    p2 = pred2.reshape(-1, _C)
    pad = jnp.ones((10,), jnp.float32)
    anch = jnp.concatenate([
        anchors0.reshape(-1), pad,
        anchors1.reshape(-1), pad,
        anchors2.reshape(-1), pad,
    ])
    parts = _sc_partials(p0, p1, p2, targets, anch)
    t = parts.reshape(_NW, 6, 16).sum(axis=(0, 2))
    lbox = (t[0] / jnp.maximum(t[3], 1.0)
            + t[1] / jnp.maximum(t[4], 1.0)
            + t[2] / jnp.maximum(t[5], 1.0))
    return lbox.astype(jnp.float32)
